# baseline (speedup 1.0000x reference)
import functools

import jax
import jax.numpy as jnp
from jax import lax
from jax.experimental import pallas as pl
from jax.experimental.pallas import tpu as pltpu
from jax.experimental.pallas import tpu_sc as plsc

_B = 16384
_C = 8
_NS = 16
_BPW = _B // _NS
_L = 16
_STEPS = _BPW // _L


def _sc_body(data_hbm, mask_hbm, out_hbm, data_v, mask_v, out_v, dsem, msem):
    base = lax.axis_index("s") * _BPW
    dcp = pltpu.async_copy(data_hbm.at[pl.ds(base, _BPW)], data_v, dsem)
    mcp = pltpu.async_copy(mask_hbm.at[pl.ds(base * _C, _BPW * _C)], mask_v, msem)
    dcp.wait()
    mcp.wait()

    bvec = lax.iota(jnp.int32, _L) * _C - 1

    @pl.loop(0, _STEPS, unroll=8)
    def _gather(j):
        d = data_v[pl.ds(j * _L, _L)]
        flat = d + (bvec + j * (_L * _C))
        out_v[pl.ds(j * _L, _L)] = plsc.load_gather(mask_v, [flat])

    pltpu.sync_copy(out_v, out_hbm.at[pl.ds(base, _BPW)])


_sc_call = functools.partial(
    pl.kernel,
    out_type=jax.ShapeDtypeStruct((_B,), jnp.float32),
    mesh=plsc.VectorSubcoreMesh(
        core_axis_name="c", subcore_axis_name="s", num_cores=1
    ),
    compiler_params=pltpu.CompilerParams(needs_layout_passes=False),
    scratch_types=[
        pltpu.VMEM((_BPW,), jnp.int32),
        pltpu.VMEM((_BPW * _C,), jnp.float32),
        pltpu.VMEM((_BPW,), jnp.float32),
        pltpu.SemaphoreType.DMA,
        pltpu.SemaphoreType.DMA,
    ],
)(_sc_body)


def kernel(data, color_mask):
    return _sc_call(data.astype(jnp.int32), color_mask.reshape(-1))


# speedup vs baseline: 1.0094x; 1.0094x over previous
import functools

import jax
import jax.numpy as jnp
from jax import lax
from jax.experimental import pallas as pl
from jax.experimental.pallas import tpu as pltpu
from jax.experimental.pallas import tpu_sc as plsc

_B = 16384
_C = 8
_NS = 16
_BPW = _B // _NS
_L = 16
_STEPS = _BPW // _L


def _sc_body(data_hbm, mask_hbm, out_hbm, data_v, mask_v, out_v, dsem, msem):
    base = lax.axis_index("s") * _BPW
    dcp = pltpu.async_copy(data_hbm.at[pl.ds(base, _BPW)], data_v, dsem)
    mcp = pltpu.async_copy(mask_hbm.at[pl.ds(base * _C, _BPW * _C)], mask_v, msem)
    dcp.wait()
    mcp.wait()

    bvec = lax.iota(jnp.int32, _L) * _C - 1

    for j in range(_STEPS):
        d = data_v[pl.ds(j * _L, _L)]
        flat = d + (bvec + j * (_L * _C))
        out_v[pl.ds(j * _L, _L)] = plsc.load_gather(mask_v, [flat])

    pltpu.sync_copy(out_v, out_hbm.at[pl.ds(base, _BPW)])


_sc_call = functools.partial(
    pl.kernel,
    out_type=jax.ShapeDtypeStruct((_B,), jnp.float32),
    mesh=plsc.VectorSubcoreMesh(
        core_axis_name="c", subcore_axis_name="s", num_cores=1
    ),
    compiler_params=pltpu.CompilerParams(needs_layout_passes=False),
    scratch_types=[
        pltpu.VMEM((_BPW,), jnp.int32),
        pltpu.VMEM((_BPW * _C,), jnp.float32),
        pltpu.VMEM((_BPW,), jnp.float32),
        pltpu.SemaphoreType.DMA,
        pltpu.SemaphoreType.DMA,
    ],
)(_sc_body)


def kernel(data, color_mask):
    return _sc_call(data.astype(jnp.int32), color_mask.reshape(-1))


# speedup vs baseline: 1.1363x; 1.1257x over previous
import functools

import jax
import jax.numpy as jnp
from jax import lax
from jax.experimental import pallas as pl
from jax.experimental.pallas import tpu as pltpu
from jax.experimental.pallas import tpu_sc as plsc

_B = 16384
_C = 8
_NC = 2
_NS = 16
_NW = _NC * _NS
_BPW = _B // _NW
_L = 16
_STEPS = _BPW // _L


def _sc_body(data_hbm, mask_hbm, out_hbm, data_v, mask_v, out_v, dsem, msem):
    wid = lax.axis_index("s") * _NC + lax.axis_index("c")
    base = wid * _BPW
    dcp = pltpu.async_copy(data_hbm.at[pl.ds(base, _BPW)], data_v, dsem)
    mcp = pltpu.async_copy(mask_hbm.at[pl.ds(base, _BPW), :], mask_v, msem)
    dcp.wait()
    mcp.wait()

    rows = lax.iota(jnp.int32, _L)
    for j in range(_STEPS):
        cols = data_v[pl.ds(j * _L, _L)] - 1
        out_v[pl.ds(j * _L, _L)] = plsc.load_gather(
            mask_v, [rows + j * _L, cols]
        )

    pltpu.sync_copy(out_v, out_hbm.at[pl.ds(base, _BPW)])


_sc_call = functools.partial(
    pl.kernel,
    out_type=jax.ShapeDtypeStruct((_B,), jnp.float32),
    mesh=plsc.VectorSubcoreMesh(
        core_axis_name="c", subcore_axis_name="s", num_cores=_NC
    ),
    compiler_params=pltpu.CompilerParams(needs_layout_passes=False),
    scratch_types=[
        pltpu.VMEM((_BPW,), jnp.int32),
        pltpu.VMEM((_BPW, _C), jnp.float32),
        pltpu.VMEM((_BPW,), jnp.float32),
        pltpu.SemaphoreType.DMA,
        pltpu.SemaphoreType.DMA,
    ],
)(_sc_body)


def kernel(data, color_mask):
    return _sc_call(data, color_mask)
